# baseline (speedup 1.0000x reference)
import functools

import jax
import jax.numpy as jnp
from jax import lax
from jax.experimental import pallas as pl
from jax.experimental.pallas import tpu as pltpu
from jax.experimental.pallas import tpu_sc as plsc

B = 64
H = 512
W = 512
HO = 256
WO = 1024

NUM_CORES = 2
NUM_SUBCORES = 16
NW = NUM_CORES * NUM_SUBCORES
TASKS = B * 16
TPW = TASKS // NW

_mesh = plsc.VectorSubcoreMesh(
    core_axis_name="c", subcore_axis_name="s",
    num_cores=NUM_CORES, num_subcores=NUM_SUBCORES)


@functools.partial(
    pl.kernel,
    out_type=jax.ShapeDtypeStruct((B, HO, WO), jnp.float32),
    mesh=_mesh,
    compiler_params=pltpu.CompilerParams(use_tc_tiling_on_sc=False, needs_layout_passes=False),
    scratch_types=[
        pltpu.VMEM((32, W), jnp.float32),
        pltpu.VMEM((16, WO), jnp.float32),
    ],
)
def _unweave(in_hbm, out_hbm, inbuf, outbuf):
    cid = lax.axis_index("c")
    sid = lax.axis_index("s")
    wid = sid * NUM_CORES + cid

    lane = lax.iota(jnp.int32, 16)
    c_lane = lane % 4
    rowpat = (c_lane // 2) * 16
    colpat = (c_lane % 2) * 16 + lane // 4

    def task_body(t, _):
        task = wid * TPW + t
        b = task // 16
        ys = task % 16
        pltpu.sync_copy(in_hbm.at[b, pl.ds(ys * 32, 32), :], inbuf)

        def yi_body(yi, _):
            rowidx = rowpat + yi

            def k_body(k, _):
                colidx = colpat + (k % 4) * 4 + (k // 4) * 32
                vals = plsc.load_gather(inbuf, [rowidx, colidx])
                outbuf[yi, pl.ds(k * 16, 16)] = vals
                return 0

            lax.fori_loop(0, 64, k_body, 0, unroll=8)
            return 0

        lax.fori_loop(0, 16, yi_body, 0)
        pltpu.sync_copy(outbuf, out_hbm.at[b, pl.ds(ys * 16, 16), :])
        return 0

    lax.fori_loop(0, TPW, task_body, 0)


def kernel(image):
    img = jnp.reshape(image, (B, H, W))
    out = _unweave(img)
    return jnp.reshape(out, (B, HO, 256, 4))


# speedup vs baseline: 1.4901x; 1.4901x over previous
import functools

import jax
import jax.numpy as jnp
from jax import lax
from jax.experimental import pallas as pl
from jax.experimental.pallas import tpu as pltpu
from jax.experimental.pallas import tpu_sc as plsc

B = 64
H = 512
W = 512
BAND = 32 * W

NUM_CORES = 2
NUM_SUBCORES = 16
NW = NUM_CORES * NUM_SUBCORES
TASKS = B * 16
TPW = TASKS // NW

_mesh = plsc.VectorSubcoreMesh(
    core_axis_name="c", subcore_axis_name="s",
    num_cores=NUM_CORES, num_subcores=NUM_SUBCORES)


@functools.partial(
    pl.kernel,
    out_type=jax.ShapeDtypeStruct((B, 16, BAND), jnp.float32),
    mesh=_mesh,
    compiler_params=pltpu.CompilerParams(
        use_tc_tiling_on_sc=False, needs_layout_passes=False),
    scratch_types=[
        pltpu.VMEM((BAND,), jnp.float32),
        pltpu.VMEM((BAND,), jnp.float32),
        pltpu.VMEM((BAND,), jnp.float32),
        pltpu.VMEM((BAND,), jnp.float32),
        pltpu.SemaphoreType.DMA,
        pltpu.SemaphoreType.DMA,
        pltpu.SemaphoreType.DMA,
        pltpu.SemaphoreType.DMA,
    ],
)
def _unweave(in_hbm, out_hbm, in_a, in_b, out_a, out_b, si_a, si_b, so_a, so_b):
    cid = lax.axis_index("c")
    sid = lax.axis_index("s")
    wid = sid * NUM_CORES + cid

    lane = lax.iota(jnp.int32, 16)
    c_lane = lane % 4
    flatpat = (c_lane // 2) * (16 * W) + (c_lane % 2) * 16 + lane // 4

    ins = [in_a, in_b]
    outs = [out_a, out_b]
    isems = [si_a, si_b]
    osems = [so_a, so_b]

    def hbm_in(t):
        task = wid * TPW + t
        return in_hbm.at[task // 16, task % 16]

    def hbm_out(t):
        task = wid * TPW + t
        return out_hbm.at[task // 16, task % 16]

    in_desc = [None, None]
    out_desc = [None, None]
    in_desc[0] = pltpu.async_copy(hbm_in(0), ins[0], isems[0])
    for t in range(TPW):
        sl = t % 2
        if t + 1 < TPW:
            in_desc[1 - sl] = pltpu.async_copy(
                hbm_in(t + 1), ins[1 - sl], isems[1 - sl])
        in_desc[sl].wait()
        if out_desc[sl] is not None:
            out_desc[sl].wait()
        ibuf = ins[sl]
        obuf = outs[sl]

        @plsc.parallel_loop(0, 1024, step=1, unroll=8)
        def _chunk(m):
            off = (m >> 6) * W + (m & 3) * 4 + ((m >> 2) & 15) * 32
            vals = plsc.load_gather(ibuf, [flatpat + off])
            obuf[pl.ds(m * 16, 16)] = vals

        out_desc[sl] = pltpu.async_copy(obuf, hbm_out(t), osems[sl])
    out_desc[0].wait()
    out_desc[1].wait()


def kernel(image):
    img = jnp.reshape(image, (B, 16, BAND))
    out = _unweave(img)
    return jnp.reshape(out, (B, 256, 256, 4))
